# baseline (speedup 1.0000x reference)
import functools

import jax
import jax.numpy as jnp
from jax import lax
from jax.experimental import pallas as pl
from jax.experimental.pallas import tpu as pltpu
from jax.experimental.pallas import tpu_sc as plsc

M = 4096
N = 4096
V = 4
TXS = 16
TYS = 16
LUT_SIZE = 512
W_SCALE = 0.02

NC, NS, L = 2, 16, 16
NW = NC * NS
NBT = M // TXS
TILES_PER_W = NBT // NW
ROWS_PER_W = TILES_PER_W * TXS
WROWS = N // TYS
WCOLS = TXS * V
WPAD = WCOLS + 4
QUADS = N // (V * L)
QUAD_UNROLL = 8


def _sc_body(walks_hbm, lut_hbm, sr_hbm, sl_hbm, out_hbm,
             lutbuf, srbuf, tabbuf, pvbuf, wb0, wb1, slbuf, rb0, rb1,
             wsem, rsem):
    wid = lax.axis_index("s") * NC + lax.axis_index("c")
    lane = lax.iota(jnp.int32, L)
    wrow_pat = lane // 4
    wcol_pat = lane % 4
    vpat = [(lane + lane // 4 + r) % 4 for r in range(V)]
    spat = [lane * 4 + vpat[r] for r in range(V)]

    pltpu.sync_copy(lut_hbm, lutbuf)
    pltpu.sync_copy(sr_hbm, srbuf)
    pltpu.sync_copy(sl_hbm.at[pl.ds(wid * ROWS_PER_W, ROWS_PER_W)], slbuf)

    for p in range(16):
        bits = (jnp.full((L,), p, jnp.int32) >> (lane % 4)) & 1
        sv = jnp.where(bits == 1, -W_SCALE, W_SCALE)

        @plsc.parallel_loop(0, LUT_SIZE * V // L, 1, unroll=8)
        def _build_tab(i, p=p, sv=sv):
            tabbuf[pl.ds(p * 2048 + i * L, L)] = lutbuf[pl.ds(i * L, L)] * sv

    @plsc.parallel_loop(0, N // (V * L), 1, unroll=4)
    def _build_pv(i):
        kidx = (i * L + lane) * 4
        acc = jnp.zeros((L,), jnp.int32)
        for v in range(V):
            g = plsc.load_gather(srbuf, [kidx + v])
            acc = acc + jnp.where(g < 0.0, 1 << v, 0).astype(jnp.int32)
        pvbuf[pl.ds(i * L, L)] = acc * 2048

    for t in range(TILES_PER_W):
        bt = wid * TILES_PER_W + t
        wb = wb0 if t % 2 == 0 else wb1
        if t == 0:
            pltpu.sync_copy(walks_hbm.at[bt], wb.at[:, pl.ds(0, WCOLS)])
        if t + 1 < TILES_PER_W:
            pltpu.async_copy(walks_hbm.at[bt + 1],
                             (wb1 if t % 2 == 0 else wb0).at[:, pl.ds(0, WCOLS)],
                             wsem)

        def _do_pair(i, c, t=t, wb=wb):
            for sub in range(2):
                tx = i * 2 + sub
                rg = t * TXS + tx
                rb = rb0 if sub == 0 else rb1

                @pl.when(rg >= 2)
                def _wait_prev():
                    pltpu.make_async_copy(rb0, out_hbm.at[0], rsem).wait()

                slsplat = plsc.load_gather(slbuf,
                                           [jnp.full((L,), rg, jnp.int32)])
                wcol = wcol_pat + tx * 4

                @plsc.parallel_loop(0, QUADS, 1, unroll=QUAD_UNROLL)
                def _quad(jq):
                    widx = plsc.load_gather(wb, [wrow_pat + jq * 4, wcol])
                    pvec = pvbuf[pl.ds(jq * L, L)]
                    cbase = pvec + widx * 4
                    for r in range(V):
                        g = plsc.load_gather(tabbuf, [cbase + vpat[r]])
                        plsc.store_scatter(rb, [spat[r] + jq * 64],
                                           g * slsplat)

                pltpu.async_copy(rb, out_hbm.at[wid * ROWS_PER_W + rg], rsem)
            return c
        lax.fori_loop(0, TXS // 2, _do_pair, 0)

        if t + 1 < TILES_PER_W:
            pltpu.make_async_copy(walks_hbm.at[0],
                                  wb0.at[:, pl.ds(0, WCOLS)], wsem).wait()

    pltpu.make_async_copy(rb0, out_hbm.at[0], rsem).wait()
    pltpu.make_async_copy(rb0, out_hbm.at[0], rsem).wait()


@jax.jit
def _sc_dequant(walks, lut_flat, sign_r, sign_l):
    mesh = plsc.VectorSubcoreMesh(core_axis_name="c", subcore_axis_name="s",
                                  num_cores=NC, num_subcores=NS)
    f = pl.kernel(
        _sc_body,
        out_type=jax.ShapeDtypeStruct((M, N), jnp.float32),
        mesh=mesh,
        compiler_params=pltpu.CompilerParams(needs_layout_passes=False,
                                             use_tc_tiling_on_sc=False),
        scratch_types=[
            pltpu.VMEM((LUT_SIZE * V,), jnp.float32),
            pltpu.VMEM((N,), jnp.float32),
            pltpu.VMEM((16 * LUT_SIZE * V,), jnp.float32),
            pltpu.VMEM((N // V,), jnp.int32),
            pltpu.VMEM((WROWS, WPAD), jnp.int32),
            pltpu.VMEM((WROWS, WPAD), jnp.int32),
            pltpu.VMEM((ROWS_PER_W,), jnp.float32),
            pltpu.VMEM((N,), jnp.float32),
            pltpu.VMEM((N,), jnp.float32),
            pltpu.SemaphoreType.DMA,
            pltpu.SemaphoreType.DMA,
        ],
    )
    return f(walks, lut_flat, sign_r, sign_l)


def kernel(walks, lut, sign_l, sign_r):
    walks = walks.astype(jnp.int32).reshape(NBT, WROWS, WCOLS)
    lut_flat = lut.reshape(LUT_SIZE * V)
    return _sc_dequant(walks, lut_flat, sign_r, sign_l)


# speedup vs baseline: 1.7406x; 1.7406x over previous
import functools

import jax
import jax.numpy as jnp
from jax import lax
from jax.experimental import pallas as pl
from jax.experimental.pallas import tpu as pltpu
from jax.experimental.pallas import tpu_sc as plsc

M = 4096
N = 4096
V = 4
TXS = 16
TYS = 16
LUT_SIZE = 512
W_SCALE = 0.02

NC, NS, L = 2, 16, 16
NW = NC * NS
NBT = M // TXS
TILES_PER_W = NBT // NW
ROWS_PER_W = TILES_PER_W * TXS
WPT = N * TXS // V
QUADS = N // (V * L)
QUAD_UNROLL = 8


def _sc_body(walks_hbm, lut_hbm, sr_hbm, sl_hbm, out_hbm,
             lutbuf, srbuf, srowv, lutpair, wb0, wb1, slbuf, rb0, rb1,
             wsem, rsem):
    wid = lax.axis_index("s") * NC + lax.axis_index("c")
    lane = lax.iota(jnp.int32, L)
    pat_w = (lane // 4) * 64 + (lane % 4)
    pat_s = lane * 4

    pltpu.sync_copy(lut_hbm, lutbuf)
    pltpu.sync_copy(sr_hbm, srbuf)
    pltpu.sync_copy(sl_hbm.at[pl.ds(wid * ROWS_PER_W, ROWS_PER_W)], slbuf)

    def _build_srow(i, c):
        kidx = i * L + lane
        for v in range(V):
            g = plsc.load_gather(srbuf, [kidx * 4 + v])
            srowv[v, pl.ds(i * L, L)] = g
        return c
    lax.fori_loop(0, N // (V * L), _build_srow, 0)

    hmask = jnp.full((L,), -65536, jnp.int32)

    @plsc.parallel_loop(0, LUT_SIZE * V // (2 * L), 1, unroll=4)
    def _build_pair(i):
        e = (i * L + lane) * 2
        blo = plsc.bitcast(plsc.load_gather(lutbuf, [e]) * W_SCALE, jnp.int32)
        bhi = plsc.bitcast(plsc.load_gather(lutbuf, [e + 1]) * W_SCALE,
                           jnp.int32)
        lo16 = lax.shift_right_logical(blo + 0x8000, 16)
        hi16 = (bhi + 0x8000) & hmask
        lutpair[pl.ds(i * L, L)] = lo16 | hi16

    for t in range(TILES_PER_W):
        bt = wid * TILES_PER_W + t
        wb = wb0 if t % 2 == 0 else wb1
        if t == 0:
            pltpu.sync_copy(walks_hbm.at[pl.ds(bt * WPT, WPT)], wb)
        if t + 1 < TILES_PER_W:
            pltpu.async_copy(walks_hbm.at[pl.ds((bt + 1) * WPT, WPT)],
                             wb1 if t % 2 == 0 else wb0, wsem)

        def _do_pair(i, c, t=t, wb=wb):
            for sub in range(2):
                tx = i * 2 + sub
                rg = t * TXS + tx
                rb = rb0 if sub == 0 else rb1

                @pl.when(rg >= 2)
                def _wait_prev():
                    pltpu.make_async_copy(rb0, out_hbm.at[0], rsem).wait()

                slsplat = plsc.load_gather(slbuf,
                                           [jnp.full((L,), rg, jnp.int32)])

                @plsc.parallel_loop(0, QUADS, 1, unroll=QUAD_UNROLL)
                def _quad(jq):
                    widx = plsc.load_gather(wb, [pat_w + (jq * 256 + tx * 4)])
                    pbase = widx * 2
                    for h in range(2):
                        gp = plsc.load_gather(lutpair, [pbase + h])
                        glo = plsc.bitcast(lax.shift_left(gp, 16), jnp.float32)
                        ghi = plsc.bitcast(gp & hmask, jnp.float32)
                        for v, g in ((2 * h, glo), (2 * h + 1, ghi)):
                            s = srowv[v, pl.ds(jq * L, L)]
                            plsc.store_scatter(rb, [pat_s + (jq * 64 + v)],
                                               g * s * slsplat)

                pltpu.async_copy(rb, out_hbm.at[wid * ROWS_PER_W + rg], rsem)
            return c
        lax.fori_loop(0, TXS // 2, _do_pair, 0)

        if t + 1 < TILES_PER_W:
            pltpu.make_async_copy(walks_hbm.at[pl.ds(0, WPT)], wb0, wsem).wait()

    pltpu.make_async_copy(rb0, out_hbm.at[0], rsem).wait()
    pltpu.make_async_copy(rb0, out_hbm.at[0], rsem).wait()


@jax.jit
def _sc_dequant(walks, lut_flat, sign_r, sign_l):
    mesh = plsc.VectorSubcoreMesh(core_axis_name="c", subcore_axis_name="s",
                                  num_cores=NC, num_subcores=NS)
    f = pl.kernel(
        _sc_body,
        out_type=jax.ShapeDtypeStruct((M, N), jnp.float32),
        mesh=mesh,
        compiler_params=pltpu.CompilerParams(needs_layout_passes=False),
        scratch_types=[
            pltpu.VMEM((LUT_SIZE * V,), jnp.float32),
            pltpu.VMEM((N,), jnp.float32),
            pltpu.VMEM((V, N // V), jnp.float32),
            pltpu.VMEM((LUT_SIZE * 2,), jnp.int32),
            pltpu.VMEM((WPT,), jnp.int32),
            pltpu.VMEM((WPT,), jnp.int32),
            pltpu.VMEM((ROWS_PER_W,), jnp.float32),
            pltpu.VMEM((N,), jnp.float32),
            pltpu.VMEM((N,), jnp.float32),
            pltpu.SemaphoreType.DMA,
            pltpu.SemaphoreType.DMA,
        ],
    )
    return f(walks, lut_flat, sign_r, sign_l)


def kernel(walks, lut, sign_l, sign_r):
    walks = walks.astype(jnp.int32)
    lut_flat = lut.reshape(LUT_SIZE * V)
    return _sc_dequant(walks, lut_flat, sign_r, sign_l)


# speedup vs baseline: 1.9504x; 1.1206x over previous
import functools

import jax
import jax.numpy as jnp
from jax import lax
from jax.experimental import pallas as pl
from jax.experimental.pallas import tpu as pltpu
from jax.experimental.pallas import tpu_sc as plsc

M = 4096
N = 4096
V = 4
TXS = 16
TYS = 16
LUT_SIZE = 512
W_SCALE = 0.02

NC, NS, L = 2, 16, 16
NW = NC * NS
NBT = M // TXS
TILES_PER_W = NBT // NW
ROWS_PER_W = TILES_PER_W * TXS
WPT = N * TXS // V
QUADS = N // (V * L)
QUAD_UNROLL = 8


def _sc_body(walks_hbm, lut_hbm, sr_hbm, sl_hbm, out_hbm,
             lutbuf, srbuf, tabp, pvbuf, wb0, wb1, slbuf, rb0, rb1,
             wsem, rsem):
    wid = lax.axis_index("s") * NC + lax.axis_index("c")
    lane = lax.iota(jnp.int32, L)
    pat_w = (lane // 4) * 64 + (lane % 4)
    pat_s = lane * 4

    pltpu.sync_copy(lut_hbm, lutbuf)
    pltpu.sync_copy(sr_hbm, srbuf)
    pltpu.sync_copy(sl_hbm.at[pl.ds(wid * ROWS_PER_W, ROWS_PER_W)], slbuf)

    hmask = jnp.full((L,), -65536, jnp.int32)
    for p in range(16):
        blo_p = (jnp.full((L,), p, jnp.int32) >> (2 * (lane % 2))) & 1
        bhi_p = (jnp.full((L,), p, jnp.int32) >> (2 * (lane % 2) + 1)) & 1
        slo = jnp.where(blo_p == 1, -W_SCALE, W_SCALE)
        shi = jnp.where(bhi_p == 1, -W_SCALE, W_SCALE)

        @plsc.parallel_loop(0, LUT_SIZE * 2 // L, 1, unroll=4)
        def _build_pair(i, p=p, slo=slo, shi=shi):
            e = (i * L + lane) * 2
            blo = plsc.bitcast(plsc.load_gather(lutbuf, [e]) * slo, jnp.int32)
            bhi = plsc.bitcast(plsc.load_gather(lutbuf, [e + 1]) * shi,
                               jnp.int32)
            lo16 = lax.shift_right_logical(blo + 0x8000, 16)
            hi16 = (bhi + 0x8000) & hmask
            tabp[pl.ds(p * 1024 + i * L, L)] = lo16 | hi16

    @plsc.parallel_loop(0, N // (V * L), 1, unroll=4)
    def _build_pv(i):
        kidx = (i * L + lane) * 4
        acc = jnp.zeros((L,), jnp.int32)
        for v in range(V):
            g = plsc.load_gather(srbuf, [kidx + v])
            acc = acc + jnp.where(g < 0.0, 1 << v, 0).astype(jnp.int32)
        pvbuf[pl.ds(i * L, L)] = acc * 1024

    for t in range(TILES_PER_W):
        bt = wid * TILES_PER_W + t
        wb = wb0 if t % 2 == 0 else wb1
        if t == 0:
            pltpu.sync_copy(walks_hbm.at[pl.ds(bt * WPT, WPT)], wb)
        if t + 1 < TILES_PER_W:
            pltpu.async_copy(walks_hbm.at[pl.ds((bt + 1) * WPT, WPT)],
                             wb1 if t % 2 == 0 else wb0, wsem)

        def _do_pair(i, c, t=t, wb=wb):
            for sub in range(2):
                tx = i * 2 + sub
                rg = t * TXS + tx
                rb = rb0 if sub == 0 else rb1

                @pl.when(rg >= 2)
                def _wait_prev():
                    pltpu.make_async_copy(rb0, out_hbm.at[0], rsem).wait()

                slsplat = plsc.load_gather(slbuf,
                                           [jnp.full((L,), rg, jnp.int32)])

                @plsc.parallel_loop(0, QUADS, 1, unroll=QUAD_UNROLL)
                def _quad(jq):
                    widx = plsc.load_gather(wb, [pat_w + (jq * 256 + tx * 4)])
                    pvec = pvbuf[pl.ds(jq * L, L)]
                    pbase = pvec + widx * 2
                    for h in range(2):
                        gp = plsc.load_gather(tabp, [pbase + h])
                        glo = plsc.bitcast(lax.shift_left(gp, 16), jnp.float32)
                        ghi = plsc.bitcast(gp & hmask, jnp.float32)
                        for v, g in ((2 * h, glo), (2 * h + 1, ghi)):
                            plsc.store_scatter(rb, [pat_s + (jq * 64 + v)],
                                               g * slsplat)

                pltpu.async_copy(rb, out_hbm.at[wid * ROWS_PER_W + rg], rsem)
            return c
        lax.fori_loop(0, TXS // 2, _do_pair, 0)

        if t + 1 < TILES_PER_W:
            pltpu.make_async_copy(walks_hbm.at[pl.ds(0, WPT)], wb0, wsem).wait()

    pltpu.make_async_copy(rb0, out_hbm.at[0], rsem).wait()
    pltpu.make_async_copy(rb0, out_hbm.at[0], rsem).wait()


@jax.jit
def _sc_dequant(walks, lut_flat, sign_r, sign_l):
    mesh = plsc.VectorSubcoreMesh(core_axis_name="c", subcore_axis_name="s",
                                  num_cores=NC, num_subcores=NS)
    f = pl.kernel(
        _sc_body,
        out_type=jax.ShapeDtypeStruct((M, N), jnp.float32),
        mesh=mesh,
        compiler_params=pltpu.CompilerParams(needs_layout_passes=False),
        scratch_types=[
            pltpu.VMEM((LUT_SIZE * V,), jnp.float32),
            pltpu.VMEM((N,), jnp.float32),
            pltpu.VMEM((16 * LUT_SIZE * 2,), jnp.int32),
            pltpu.VMEM((N // V,), jnp.int32),
            pltpu.VMEM((WPT,), jnp.int32),
            pltpu.VMEM((WPT,), jnp.int32),
            pltpu.VMEM((ROWS_PER_W,), jnp.float32),
            pltpu.VMEM((N,), jnp.float32),
            pltpu.VMEM((N,), jnp.float32),
            pltpu.SemaphoreType.DMA,
            pltpu.SemaphoreType.DMA,
        ],
    )
    return f(walks, lut_flat, sign_r, sign_l)


def kernel(walks, lut, sign_l, sign_r):
    walks = walks.astype(jnp.int32)
    lut_flat = lut.reshape(LUT_SIZE * V)
    return _sc_dequant(walks, lut_flat, sign_r, sign_l)


# speedup vs baseline: 1.9665x; 1.0082x over previous
import functools

import jax
import jax.numpy as jnp
from jax import lax
from jax.experimental import pallas as pl
from jax.experimental.pallas import tpu as pltpu
from jax.experimental.pallas import tpu_sc as plsc

M = 4096
N = 4096
V = 4
TXS = 16
TYS = 16
LUT_SIZE = 512
W_SCALE = 0.02

NC, NS, L = 2, 16, 16
NW = NC * NS
NBT = M // TXS
TILES_PER_W = NBT // NW
ROWS_PER_W = TILES_PER_W * TXS
WPT = N * TXS // V
QUADS = N // (V * L)
QUAD_UNROLL = 8


def _sc_body(walks_hbm, lut_hbm, sr_hbm, sl_hbm, out_hbm,
             lutbuf, srbuf, tabp, pvbuf, wb0, wb1, slbuf, rb0, rb1,
             wsem, rsem):
    wid = lax.axis_index("s") * NC + lax.axis_index("c")
    lane = lax.iota(jnp.int32, L)
    pat_w = (lane // 4) * 64 + (lane % 4)
    pat_s = lane * 4

    pltpu.sync_copy(lut_hbm, lutbuf)
    pltpu.sync_copy(sr_hbm, srbuf)
    pltpu.sync_copy(sl_hbm.at[pl.ds(wid * ROWS_PER_W, ROWS_PER_W)], slbuf)

    hmask = jnp.full((L,), -65536, jnp.int32)
    for p in range(32):
        sl_neg = -1.0 if p >= 16 else 1.0
        blo_p = (jnp.full((L,), p, jnp.int32) >> (2 * (lane % 2))) & 1
        bhi_p = (jnp.full((L,), p, jnp.int32) >> (2 * (lane % 2) + 1)) & 1
        slo = jnp.where(blo_p == 1, -W_SCALE * sl_neg, W_SCALE * sl_neg)
        shi = jnp.where(bhi_p == 1, -W_SCALE * sl_neg, W_SCALE * sl_neg)

        @plsc.parallel_loop(0, LUT_SIZE * 2 // L, 1, unroll=4)
        def _build_pair(i, p=p, slo=slo, shi=shi):
            e = (i * L + lane) * 2
            blo = plsc.bitcast(plsc.load_gather(lutbuf, [e]) * slo, jnp.int32)
            bhi = plsc.bitcast(plsc.load_gather(lutbuf, [e + 1]) * shi,
                               jnp.int32)
            lo16 = lax.shift_right_logical(blo + 0x8000, 16)
            hi16 = (bhi + 0x8000) & hmask
            tabp[pl.ds(p * 1024 + i * L, L)] = lo16 | hi16

    @plsc.parallel_loop(0, N // (V * L), 1, unroll=4)
    def _build_pv(i):
        kidx = (i * L + lane) * 4
        acc = jnp.zeros((L,), jnp.int32)
        for v in range(V):
            g = plsc.load_gather(srbuf, [kidx + v])
            acc = acc + jnp.where(g < 0.0, 1 << v, 0).astype(jnp.int32)
        pvbuf[pl.ds(i * L, L)] = acc * 1024

    for t in range(TILES_PER_W):
        bt = wid * TILES_PER_W + t
        wb = wb0 if t % 2 == 0 else wb1
        if t == 0:
            pltpu.sync_copy(walks_hbm.at[pl.ds(bt * WPT, WPT)], wb)
        if t + 1 < TILES_PER_W:
            pltpu.async_copy(walks_hbm.at[pl.ds((bt + 1) * WPT, WPT)],
                             wb1 if t % 2 == 0 else wb0, wsem)

        def _do_pair(i, c, t=t, wb=wb):
            for sub in range(2):
                tx = i * 2 + sub
                rg = t * TXS + tx
                rb = rb0 if sub == 0 else rb1

                @pl.when(rg >= 2)
                def _wait_prev():
                    pltpu.make_async_copy(rb0, out_hbm.at[0], rsem).wait()

                slsplat = plsc.load_gather(slbuf,
                                           [jnp.full((L,), rg, jnp.int32)])
                slofs = jnp.where(slsplat < 0.0, 16 * 1024, 0).astype(jnp.int32)

                @plsc.parallel_loop(0, QUADS, 1, unroll=QUAD_UNROLL)
                def _quad(jq):
                    widx = plsc.load_gather(wb, [pat_w + (jq * 256 + tx * 4)])
                    pvec = pvbuf[pl.ds(jq * L, L)]
                    pbase = pvec + slofs + widx * 2
                    for h in range(2):
                        gp = plsc.load_gather(tabp, [pbase + h])
                        glo = plsc.bitcast(lax.shift_left(gp, 16), jnp.float32)
                        ghi = plsc.bitcast(gp & hmask, jnp.float32)
                        for v, g in ((2 * h, glo), (2 * h + 1, ghi)):
                            plsc.store_scatter(rb, [pat_s + (jq * 64 + v)], g)

                pltpu.async_copy(rb, out_hbm.at[wid * ROWS_PER_W + rg], rsem)
            return c
        lax.fori_loop(0, TXS // 2, _do_pair, 0)

        if t + 1 < TILES_PER_W:
            pltpu.make_async_copy(walks_hbm.at[pl.ds(0, WPT)], wb0, wsem).wait()

    pltpu.make_async_copy(rb0, out_hbm.at[0], rsem).wait()
    pltpu.make_async_copy(rb0, out_hbm.at[0], rsem).wait()


@jax.jit
def _sc_dequant(walks, lut_flat, sign_r, sign_l):
    mesh = plsc.VectorSubcoreMesh(core_axis_name="c", subcore_axis_name="s",
                                  num_cores=NC, num_subcores=NS)
    f = pl.kernel(
        _sc_body,
        out_type=jax.ShapeDtypeStruct((M, N), jnp.float32),
        mesh=mesh,
        compiler_params=pltpu.CompilerParams(needs_layout_passes=False),
        scratch_types=[
            pltpu.VMEM((LUT_SIZE * V,), jnp.float32),
            pltpu.VMEM((N,), jnp.float32),
            pltpu.VMEM((32 * LUT_SIZE * 2,), jnp.int32),
            pltpu.VMEM((N // V,), jnp.int32),
            pltpu.VMEM((WPT,), jnp.int32),
            pltpu.VMEM((WPT,), jnp.int32),
            pltpu.VMEM((ROWS_PER_W,), jnp.float32),
            pltpu.VMEM((N,), jnp.float32),
            pltpu.VMEM((N,), jnp.float32),
            pltpu.SemaphoreType.DMA,
            pltpu.SemaphoreType.DMA,
        ],
    )
    return f(walks, lut_flat, sign_r, sign_l)


def kernel(walks, lut, sign_l, sign_r):
    walks = walks.astype(jnp.int32)
    lut_flat = lut.reshape(LUT_SIZE * V)
    return _sc_dequant(walks, lut_flat, sign_r, sign_l)


# speedup vs baseline: 1.9858x; 1.0098x over previous
import functools

import jax
import jax.numpy as jnp
from jax import lax
from jax.experimental import pallas as pl
from jax.experimental.pallas import tpu as pltpu
from jax.experimental.pallas import tpu_sc as plsc

M = 4096
N = 4096
V = 4
TXS = 16
TYS = 16
LUT_SIZE = 512
W_SCALE = 0.02

NC, NS, L = 2, 16, 16
NW = NC * NS
NBT = M // TXS
TILES_PER_W = NBT // NW
ROWS_PER_W = TILES_PER_W * TXS
WPT = N * TXS // V
QUADS = N // (V * L)
QUAD_UNROLL = 8


def _sc_body(walks_hbm, lut_hbm, sr_hbm, sl_hbm, out_hbm,
             lutbuf, srbuf, tabp, pvbuf, wb0, wb1, slbuf, rb0, rb1,
             wsem, rsem):
    wid = lax.axis_index("s") * NC + lax.axis_index("c")
    lane = lax.iota(jnp.int32, L)
    pat_w = (lane // 4) * 64 + (lane % 4)
    pat_s = lane * 4

    pltpu.async_copy(walks_hbm.at[pl.ds(wid * TILES_PER_W * WPT, WPT)],
                     wb0, wsem)
    pltpu.sync_copy(lut_hbm, lutbuf)
    pltpu.sync_copy(sr_hbm, srbuf)
    pltpu.sync_copy(sl_hbm.at[pl.ds(wid * ROWS_PER_W, ROWS_PER_W)], slbuf)

    hmask = jnp.full((L,), -65536, jnp.int32)
    for p in range(32):
        sl_neg = -1.0 if p >= 16 else 1.0
        blo_p = (jnp.full((L,), p, jnp.int32) >> (2 * (lane % 2))) & 1
        bhi_p = (jnp.full((L,), p, jnp.int32) >> (2 * (lane % 2) + 1)) & 1
        slo = jnp.where(blo_p == 1, -W_SCALE * sl_neg, W_SCALE * sl_neg)
        shi = jnp.where(bhi_p == 1, -W_SCALE * sl_neg, W_SCALE * sl_neg)

        @plsc.parallel_loop(0, LUT_SIZE * 2 // L, 1, unroll=4)
        def _build_pair(i, p=p, slo=slo, shi=shi):
            e = (i * L + lane) * 2
            blo = plsc.bitcast(plsc.load_gather(lutbuf, [e]) * slo, jnp.int32)
            bhi = plsc.bitcast(plsc.load_gather(lutbuf, [e + 1]) * shi,
                               jnp.int32)
            lo16 = lax.shift_right_logical(blo + 0x8000, 16)
            hi16 = (bhi + 0x8000) & hmask
            tabp[pl.ds(p * 1024 + i * L, L)] = lo16 | hi16

    @plsc.parallel_loop(0, N // (V * L), 1, unroll=4)
    def _build_pv(i):
        kidx = (i * L + lane) * 4
        acc = jnp.zeros((L,), jnp.int32)
        for v in range(V):
            g = plsc.load_gather(srbuf, [kidx + v])
            acc = acc + jnp.where(g < 0.0, 1 << v, 0).astype(jnp.int32)
        pvbuf[pl.ds(i * L, L)] = acc * 1024

    for t in range(TILES_PER_W):
        bt = wid * TILES_PER_W + t
        wb = wb0 if t % 2 == 0 else wb1
        if t == 0:
            pltpu.make_async_copy(walks_hbm.at[pl.ds(0, WPT)], wb0, wsem).wait()
        if t + 1 < TILES_PER_W:
            pltpu.async_copy(walks_hbm.at[pl.ds((bt + 1) * WPT, WPT)],
                             wb1 if t % 2 == 0 else wb0, wsem)

        def _do_pair(i, c, t=t, wb=wb):
            for sub in range(2):
                tx = i * 2 + sub
                rg = t * TXS + tx
                rb = rb0 if sub == 0 else rb1

                @pl.when(rg >= 2)
                def _wait_prev():
                    pltpu.make_async_copy(rb0, out_hbm.at[0], rsem).wait()

                slsplat = plsc.load_gather(slbuf,
                                           [jnp.full((L,), rg, jnp.int32)])
                slofs = jnp.where(slsplat < 0.0, 16 * 1024, 0).astype(jnp.int32)

                @plsc.parallel_loop(0, QUADS, 1, unroll=QUAD_UNROLL)
                def _quad(jq):
                    widx = plsc.load_gather(wb, [pat_w + (jq * 256 + tx * 4)])
                    pvec = pvbuf[pl.ds(jq * L, L)]
                    pbase = pvec + slofs + widx * 2
                    for h in range(2):
                        gp = plsc.load_gather(tabp, [pbase + h])
                        glo = plsc.bitcast(lax.shift_left(gp, 16), jnp.float32)
                        ghi = plsc.bitcast(gp & hmask, jnp.float32)
                        for v, g in ((2 * h, glo), (2 * h + 1, ghi)):
                            plsc.store_scatter(rb, [pat_s + (jq * 64 + v)], g)

                pltpu.async_copy(rb, out_hbm.at[wid * ROWS_PER_W + rg], rsem)
            return c
        lax.fori_loop(0, TXS // 2, _do_pair, 0)

        if t + 1 < TILES_PER_W:
            pltpu.make_async_copy(walks_hbm.at[pl.ds(0, WPT)], wb0, wsem).wait()

    pltpu.make_async_copy(rb0, out_hbm.at[0], rsem).wait()
    pltpu.make_async_copy(rb0, out_hbm.at[0], rsem).wait()


@jax.jit
def _sc_dequant(walks, lut_flat, sign_r, sign_l):
    mesh = plsc.VectorSubcoreMesh(core_axis_name="c", subcore_axis_name="s",
                                  num_cores=NC, num_subcores=NS)
    f = pl.kernel(
        _sc_body,
        out_type=jax.ShapeDtypeStruct((M, N), jnp.float32),
        mesh=mesh,
        compiler_params=pltpu.CompilerParams(needs_layout_passes=False,
                                             skip_device_barrier=True),
        scratch_types=[
            pltpu.VMEM((LUT_SIZE * V,), jnp.float32),
            pltpu.VMEM((N,), jnp.float32),
            pltpu.VMEM((32 * LUT_SIZE * 2,), jnp.int32),
            pltpu.VMEM((N // V,), jnp.int32),
            pltpu.VMEM((WPT,), jnp.int32),
            pltpu.VMEM((WPT,), jnp.int32),
            pltpu.VMEM((ROWS_PER_W,), jnp.float32),
            pltpu.VMEM((N,), jnp.float32),
            pltpu.VMEM((N,), jnp.float32),
            pltpu.SemaphoreType.DMA,
            pltpu.SemaphoreType.DMA,
        ],
    )
    return f(walks, lut_flat, sign_r, sign_l)


def kernel(walks, lut, sign_l, sign_r):
    walks = walks.astype(jnp.int32)
    lut_flat = lut.reshape(LUT_SIZE * V)
    return _sc_dequant(walks, lut_flat, sign_r, sign_l)
